# baseline (speedup 1.0000x reference)
import functools

import jax
import jax.numpy as jnp
from jax import lax
from jax.experimental import pallas as pl
from jax.experimental.pallas import tpu as pltpu
from jax.experimental.pallas import tpu_sc as plsc

_LOG2 = 0.6931471805599453


def _ssp(v):
    return jnp.maximum(v, 0.0) + jnp.log1p(jnp.exp(-jnp.abs(v))) - _LOG2


def _tc_pre_body(x_ref, w_ref, b_ref, y_ref):
    v = _ssp(x_ref[...])
    v = jnp.dot(v, w_ref[...], preferred_element_type=jnp.float32) + b_ref[...]
    y_ref[...] = _ssp(v)


def _tc_pre(x2, W_in2f, b_in2f, blk):
    n, d = x2.shape
    grid = (n // blk,)
    return pl.pallas_call(
        _tc_pre_body,
        grid=grid,
        in_specs=[
            pl.BlockSpec((blk, d), lambda i: (i, 0)),
            pl.BlockSpec((d, d), lambda i: (0, 0)),
            pl.BlockSpec((1, d), lambda i: (0, 0)),
        ],
        out_specs=pl.BlockSpec((blk, d), lambda i: (i, 0)),
        out_shape=jax.ShapeDtypeStruct((n, d), jnp.float32),
    )(x2, W_in2f, b_in2f.reshape(1, d))


_NC, _NS = 2, 16
_NW = _NC * _NS
_RPC = 128
_NBUF = 4


def _sc_gather(nbr2, y, d):
    n_rows = nbr2.shape[0]
    q = n_rows // _NW
    mesh = plsc.VectorSubcoreMesh(core_axis_name="c", subcore_axis_name="s")

    @functools.partial(
        pl.kernel,
        mesh=mesh,
        out_type=jax.ShapeDtypeStruct((n_rows * _RPC, d), jnp.float32),
        scratch_types=[
            pltpu.VMEM((q, _RPC), jnp.int32),
            pltpu.VMEM((_NBUF, _RPC, d), jnp.float32),
            pltpu.SemaphoreType.DMA,
            pltpu.SemaphoreType.DMA,
        ],
    )
    def gather_k(nbr_hbm, y_hbm, out_hbm, idx_v, buf_v, sem_g, sem_w):
        wid = lax.axis_index("s") * _NC + lax.axis_index("c")
        base_row = q * wid
        n_i = q
        pltpu.sync_copy(nbr_hbm.at[pl.ds(base_row, q)], idx_v)

        def start_g(i):
            pltpu.async_copy(
                y_hbm.at[idx_v.at[i]], buf_v.at[lax.rem(i, _NBUF)], sem_g)

        def wait_g(i):
            pltpu.make_async_copy(
                y_hbm.at[idx_v.at[i]], buf_v.at[lax.rem(i, _NBUF)],
                sem_g).wait()

        def start_w(i):
            pltpu.async_copy(
                buf_v.at[lax.rem(i, _NBUF)],
                out_hbm.at[pl.ds((base_row + i) * _RPC, _RPC)], sem_w)

        def wait_w():
            pltpu.make_async_copy(
                buf_v.at[0], out_hbm.at[pl.ds(base_row * _RPC, _RPC)],
                sem_w).wait()

        for k in range(_NBUF - 1):
            start_g(k)

        def body(i, carry):
            wait_g(i)

            @pl.when(i + (_NBUF - 1) < n_i)
            def _ahead():
                @pl.when(i >= 1)
                def _drain():
                    wait_w()

                start_g(i + (_NBUF - 1))

            start_w(i)
            return carry

        lax.fori_loop(0, n_i, body, 0)
        for _ in range(_NBUF):
            wait_w()

    return gather_k(nbr2, y)


def _tc_main_body(f_ref, g_ref, nm_ref, y_ref, x_ref,
                  wg_ref, w1_ref, b1_ref, w2_ref, b2_ref, w3_ref, b3_ref,
                  wd_ref, bd_ref, mask_ref, o_ref, *, blk, nbh):
    d = y_ref.shape[-1]
    sb = wg_ref.shape[0]
    f2 = f_ref[...].reshape(blk * nbh, sb)
    wf = jnp.dot(f2, wg_ref[...], preferred_element_type=jnp.float32)
    prod = (g_ref[...] * wf).reshape(blk, nbh, d)
    nm = nm_ref[...].reshape(blk, nbh)
    y2 = jnp.sum(prod * nm[..., None], axis=1)
    y = y_ref[...] + y2
    h = y
    for w_r, b_r in ((w1_ref, b1_ref), (w2_ref, b2_ref), (w3_ref, b3_ref)):
        h = _ssp(h)
        h = jnp.dot(h, w_r[...], preferred_element_type=jnp.float32) + b_r[...]
    y = y + h
    y = _ssp(y)
    y = jnp.dot(y, wd_ref[...], preferred_element_type=jnp.float32) + bd_ref[...]
    o_ref[...] = y + mask_ref[...] * x_ref[...]


def _tc_main(f3, G, nm, y, x2, W_G,
             W_res1, b_res1, W_res2, b_res2, W_res3, b_res3,
             W_dense, b_dense, mask, blk):
    n, d = x2.shape
    nbh = f3.shape[1]
    sb = f3.shape[2]
    grid = (n // blk,)
    w_spec = pl.BlockSpec((d, d), lambda i: (0, 0))
    b_spec = pl.BlockSpec((1, d), lambda i: (0, 0))
    return pl.pallas_call(
        functools.partial(_tc_main_body, blk=blk, nbh=nbh),
        grid=grid,
        in_specs=[
            pl.BlockSpec((blk, nbh, sb), lambda i: (i, 0, 0)),
            pl.BlockSpec((blk * nbh, d), lambda i: (i, 0)),
            pl.BlockSpec((1, blk, nbh), lambda i: (0, i, 0)),
            pl.BlockSpec((blk, d), lambda i: (i, 0)),
            pl.BlockSpec((blk, d), lambda i: (i, 0)),
            pl.BlockSpec((sb, d), lambda i: (0, 0)),
            w_spec, b_spec, w_spec, b_spec, w_spec, b_spec,
            w_spec, b_spec, b_spec,
        ],
        out_specs=pl.BlockSpec((blk, d), lambda i: (i, 0)),
        out_shape=jax.ShapeDtypeStruct((n, d), jnp.float32),
    )(f3, G, nm, y, x2, W_G,
      W_res1, b_res1.reshape(1, d), W_res2, b_res2.reshape(1, d),
      W_res3, b_res3.reshape(1, d), W_dense, b_dense.reshape(1, d),
      mask.reshape(1, d))


def kernel(x, r_ij, neighbors, neighbor_mask, f_ij,
           W_in2f, b_in2f, W_G,
           W_res1, b_res1, W_res2, b_res2, W_res3, b_res3,
           W_dense, b_dense, mask):
    b, n, d = x.shape
    nbh = neighbors.shape[-1]
    sb = f_ij.shape[-1]
    n_edges = b * n * nbh

    x2 = x.reshape(b * n, d)
    y = _tc_pre(x2, W_in2f, b_in2f, blk=1000)
    nbr2 = neighbors.reshape(n_edges // _RPC, _RPC)
    n_rows = nbr2.shape[0]
    pad_rows = (-n_rows) % (8 * _NW)
    if pad_rows:
        nbr2 = jnp.pad(nbr2, ((0, pad_rows), (0, 0)))

    G = _sc_gather(nbr2, y, d)

    f3 = f_ij.reshape(b * n, nbh, sb)
    out = _tc_main(f3, G, neighbor_mask, y, x2, W_G,
                   W_res1, b_res1, W_res2, b_res2, W_res3, b_res3,
                   W_dense, b_dense, mask, blk=400)
    return out.reshape(b, n, d)


# speedup vs baseline: 1.1503x; 1.1503x over previous
import functools

import jax
import jax.numpy as jnp
from jax import lax
from jax.experimental import pallas as pl
from jax.experimental.pallas import tpu as pltpu
from jax.experimental.pallas import tpu_sc as plsc

_LOG2 = 0.6931471805599453


def _ssp(v):
    return jnp.maximum(v, 0.0) + jnp.log1p(jnp.exp(-jnp.abs(v))) - _LOG2


def _tc_pre_body(x_ref, w_ref, b_ref, y_ref):
    v = _ssp(x_ref[...])
    v = jnp.dot(v, w_ref[...], preferred_element_type=jnp.float32) + b_ref[...]
    y_ref[...] = _ssp(v)


def _tc_repack_body(nbr_ref, out_ref):
    rr, _ = out_ref.shape
    _, nin, nbh = nbr_ref.shape
    inp = nbr_ref[0].astype(jnp.float32)
    r_i = lax.broadcasted_iota(jnp.int32, (rr, nin), 0)
    m_i = lax.broadcasted_iota(jnp.int32, (rr, nin), 1)
    c_i = lax.broadcasted_iota(jnp.int32, (nbh, 128), 0)
    l_i = lax.broadcasted_iota(jnp.int32, (nbh, 128), 1)
    acc = jnp.zeros((rr, 128), jnp.float32)
    for k in range(128 // nbh):
        sel = (m_i == r_i * (128 // nbh) + k).astype(jnp.float32)
        term = jnp.dot(sel, inp, preferred_element_type=jnp.float32)
        place = (l_i == nbh * k + c_i).astype(jnp.float32)
        acc = acc + jnp.dot(term, place, preferred_element_type=jnp.float32)
    out_ref[...] = acc.astype(jnp.int32)


def _tc_repack(nbr_pad, blk_rows=128):
    _, n_in, nbh = nbr_pad.shape
    n_rows = n_in * nbh // 128
    grid = (n_rows // blk_rows,)
    blk_in = blk_rows * 128 // nbh
    return pl.pallas_call(
        _tc_repack_body,
        grid=grid,
        in_specs=[pl.BlockSpec((1, blk_in, nbh), lambda i: (0, i, 0))],
        out_specs=pl.BlockSpec((blk_rows, 128), lambda i: (i, 0)),
        out_shape=jax.ShapeDtypeStruct((n_rows, 128), jnp.int32),
    )(nbr_pad)


def _tc_pre(x2, W_in2f, b_in2f, blk):
    n, d = x2.shape
    grid = (n // blk,)
    return pl.pallas_call(
        _tc_pre_body,
        grid=grid,
        in_specs=[
            pl.BlockSpec((blk, d), lambda i: (i, 0)),
            pl.BlockSpec((d, d), lambda i: (0, 0)),
            pl.BlockSpec((1, d), lambda i: (0, 0)),
        ],
        out_specs=pl.BlockSpec((blk, d), lambda i: (i, 0)),
        out_shape=jax.ShapeDtypeStruct((n, d), jnp.float32),
    )(x2, W_in2f, b_in2f.reshape(1, d))


_NC, _NS = 2, 16
_NW = _NC * _NS
_RPC = 128
_NBUF = 4


def _sc_gather(nbr2, y, d):
    n_rows = nbr2.shape[0]
    q = n_rows // _NW
    mesh = plsc.VectorSubcoreMesh(core_axis_name="c", subcore_axis_name="s")

    @functools.partial(
        pl.kernel,
        mesh=mesh,
        out_type=jax.ShapeDtypeStruct((n_rows * _RPC, d), jnp.float32),
        scratch_types=[
            pltpu.VMEM((q, _RPC), jnp.int32),
            pltpu.VMEM((_NBUF, _RPC, d), jnp.float32),
            pltpu.SemaphoreType.DMA,
            pltpu.SemaphoreType.DMA,
        ],
    )
    def gather_k(nbr_hbm, y_hbm, out_hbm, idx_v, buf_v, sem_g, sem_w):
        wid = lax.axis_index("s") * _NC + lax.axis_index("c")
        base_row = q * wid
        n_i = q
        pltpu.sync_copy(nbr_hbm.at[pl.ds(base_row, q)], idx_v)

        def start_g(i):
            pltpu.async_copy(
                y_hbm.at[idx_v.at[i]], buf_v.at[lax.rem(i, _NBUF)], sem_g)

        def wait_g(i):
            pltpu.make_async_copy(
                y_hbm.at[idx_v.at[i]], buf_v.at[lax.rem(i, _NBUF)],
                sem_g).wait()

        def start_w(i):
            pltpu.async_copy(
                buf_v.at[lax.rem(i, _NBUF)],
                out_hbm.at[pl.ds((base_row + i) * _RPC, _RPC)], sem_w)

        def wait_w():
            pltpu.make_async_copy(
                buf_v.at[0], out_hbm.at[pl.ds(base_row * _RPC, _RPC)],
                sem_w).wait()

        for k in range(_NBUF - 1):
            start_g(k)

        def body(i, carry):
            wait_g(i)

            @pl.when(i + (_NBUF - 1) < n_i)
            def _ahead():
                @pl.when(i >= 1)
                def _drain():
                    wait_w()

                start_g(i + (_NBUF - 1))

            start_w(i)
            return carry

        lax.fori_loop(0, n_i, body, 0)
        for _ in range(_NBUF):
            wait_w()

    return gather_k(nbr2, y)


def _tc_main_body(f_ref, g_ref, nm_ref, y_ref, x_ref,
                  wg_ref, w1_ref, b1_ref, w2_ref, b2_ref, w3_ref, b3_ref,
                  wd_ref, bd_ref, mask_ref, o_ref, *, blk, nbh):
    d = y_ref.shape[-1]
    sb = wg_ref.shape[0]
    f2 = f_ref[...].reshape(blk * nbh, sb)
    wf = jnp.dot(f2, wg_ref[...], preferred_element_type=jnp.float32)
    prod = (g_ref[...] * wf).reshape(blk, nbh, d)
    nm = nm_ref[...].reshape(blk, nbh)
    y2 = jnp.sum(prod * nm[..., None], axis=1)
    y = y_ref[...] + y2
    h = y
    for w_r, b_r in ((w1_ref, b1_ref), (w2_ref, b2_ref), (w3_ref, b3_ref)):
        h = _ssp(h)
        h = jnp.dot(h, w_r[...], preferred_element_type=jnp.float32) + b_r[...]
    y = y + h
    y = _ssp(y)
    y = jnp.dot(y, wd_ref[...], preferred_element_type=jnp.float32) + bd_ref[...]
    o_ref[...] = y + mask_ref[...] * x_ref[...]


def _tc_main(f3, G, nm, y, x2, W_G,
             W_res1, b_res1, W_res2, b_res2, W_res3, b_res3,
             W_dense, b_dense, mask, blk):
    n, d = x2.shape
    nbh = f3.shape[1]
    sb = f3.shape[2]
    grid = (n // blk,)
    w_spec = pl.BlockSpec((d, d), lambda i: (0, 0))
    b_spec = pl.BlockSpec((1, d), lambda i: (0, 0))
    return pl.pallas_call(
        functools.partial(_tc_main_body, blk=blk, nbh=nbh),
        grid=grid,
        in_specs=[
            pl.BlockSpec((blk, nbh, sb), lambda i: (i, 0, 0)),
            pl.BlockSpec((blk * nbh, d), lambda i: (i, 0)),
            pl.BlockSpec((1, blk, nbh), lambda i: (0, i, 0)),
            pl.BlockSpec((blk, d), lambda i: (i, 0)),
            pl.BlockSpec((blk, d), lambda i: (i, 0)),
            pl.BlockSpec((sb, d), lambda i: (0, 0)),
            w_spec, b_spec, w_spec, b_spec, w_spec, b_spec,
            w_spec, b_spec, b_spec,
        ],
        out_specs=pl.BlockSpec((blk, d), lambda i: (i, 0)),
        out_shape=jax.ShapeDtypeStruct((n, d), jnp.float32),
    )(f3, G, nm, y, x2, W_G,
      W_res1, b_res1.reshape(1, d), W_res2, b_res2.reshape(1, d),
      W_res3, b_res3.reshape(1, d), W_dense, b_dense.reshape(1, d),
      mask.reshape(1, d))


def kernel(x, r_ij, neighbors, neighbor_mask, f_ij,
           W_in2f, b_in2f, W_G,
           W_res1, b_res1, W_res2, b_res2, W_res3, b_res3,
           W_dense, b_dense, mask):
    b, n, d = x.shape
    nbh = neighbors.shape[-1]
    sb = f_ij.shape[-1]
    n_edges = b * n * nbh

    x2 = x.reshape(b * n, d)
    y = _tc_pre(x2, W_in2f, b_in2f, blk=1000)
    n_rows = n_edges // _RPC
    pad_rows = (-n_rows) % (8 * _NW)
    pad_nodes = pad_rows * _RPC // nbh
    if pad_nodes:
        nbr_pad = jnp.concatenate(
            [neighbors, neighbors[:, :pad_nodes, :]], axis=1)
    else:
        nbr_pad = neighbors
    nbr2 = _tc_repack(nbr_pad)

    G = _sc_gather(nbr2, y, d)

    f3 = f_ij.reshape(b * n, nbh, sb)
    out = _tc_main(f3, G, neighbor_mask, y, x2, W_G,
                   W_res1, b_res1, W_res2, b_res2, W_res3, b_res3,
                   W_dense, b_dense, mask, blk=400)
    return out.reshape(b, n, d)


# speedup vs baseline: 2.1617x; 1.8793x over previous
import functools

import jax
import jax.numpy as jnp
from jax import lax
from jax.experimental import pallas as pl
from jax.experimental.pallas import tpu as pltpu
from jax.experimental.pallas import tpu_sc as plsc

_LOG2 = 0.6931471805599453


def _ssp(v):
    return jnp.maximum(v, 0.0) + jnp.log1p(jnp.exp(-jnp.abs(v))) - _LOG2


def _tc_pre_body(x_ref, w_ref, b_ref, y_ref):
    v = _ssp(x_ref[...])
    v = jnp.dot(v, w_ref[...], preferred_element_type=jnp.float32) + b_ref[...]
    y_ref[...] = _ssp(v)


def _tc_repack_body(nbr_ref, out_ref):
    rr, _ = out_ref.shape
    _, nin, nbh = nbr_ref.shape
    inp = nbr_ref[0].astype(jnp.float32)
    r_i = lax.broadcasted_iota(jnp.int32, (rr, nin), 0)
    m_i = lax.broadcasted_iota(jnp.int32, (rr, nin), 1)
    c_i = lax.broadcasted_iota(jnp.int32, (nbh, 128), 0)
    l_i = lax.broadcasted_iota(jnp.int32, (nbh, 128), 1)
    acc = jnp.zeros((rr, 128), jnp.float32)
    for k in range(128 // nbh):
        sel = (m_i == r_i * (128 // nbh) + k).astype(jnp.float32)
        term = jnp.dot(sel, inp, preferred_element_type=jnp.float32,
                       precision=lax.Precision.HIGHEST)
        place = (l_i == nbh * k + c_i).astype(jnp.float32)
        acc = acc + jnp.dot(term, place, preferred_element_type=jnp.float32,
                            precision=lax.Precision.HIGHEST)
    out_ref[...] = acc.astype(jnp.int32)


def _tc_repack(nbr_pad, blk_rows=128):
    _, n_in, nbh = nbr_pad.shape
    n_rows = n_in * nbh // 128
    grid = (n_rows // blk_rows,)
    blk_in = blk_rows * 128 // nbh
    return pl.pallas_call(
        _tc_repack_body,
        grid=grid,
        in_specs=[pl.BlockSpec((1, blk_in, nbh), lambda i: (0, i, 0))],
        out_specs=pl.BlockSpec((blk_rows, 128), lambda i: (i, 0)),
        out_shape=jax.ShapeDtypeStruct((n_rows, 128), jnp.int32),
    )(nbr_pad)


def _tc_pre(x2, W_in2f, b_in2f, blk):
    n, d = x2.shape
    grid = (n // blk,)
    return pl.pallas_call(
        _tc_pre_body,
        grid=grid,
        in_specs=[
            pl.BlockSpec((blk, d), lambda i: (i, 0)),
            pl.BlockSpec((d, d), lambda i: (0, 0)),
            pl.BlockSpec((1, d), lambda i: (0, 0)),
        ],
        out_specs=pl.BlockSpec((blk, d), lambda i: (i, 0)),
        out_shape=jax.ShapeDtypeStruct((n, d), jnp.float32),
    )(x2, W_in2f, b_in2f.reshape(1, d))


_NC, _NS = 2, 16
_NW = _NC * _NS
_RPC = 128
_NBUF = 4


def _sc_gather(nbr2, y, d):
    n_rows = nbr2.shape[0]
    q = n_rows // _NW
    mesh = plsc.VectorSubcoreMesh(core_axis_name="c", subcore_axis_name="s")

    @functools.partial(
        pl.kernel,
        mesh=mesh,
        out_type=jax.ShapeDtypeStruct((n_rows * _RPC, d), jnp.float32),
        scratch_types=[
            pltpu.VMEM((q, _RPC), jnp.int32),
            pltpu.VMEM((_NBUF, _RPC, d), jnp.float32),
            pltpu.SemaphoreType.DMA,
            pltpu.SemaphoreType.DMA,
        ],
    )
    def gather_k(nbr_hbm, y_hbm, out_hbm, idx_v, buf_v, sem_g, sem_w):
        wid = lax.axis_index("s") * _NC + lax.axis_index("c")
        base_row = q * wid
        n_i = q
        pltpu.sync_copy(nbr_hbm.at[pl.ds(base_row, q)], idx_v)

        def start_g(i):
            pltpu.async_copy(
                y_hbm.at[idx_v.at[i]], buf_v.at[lax.rem(i, _NBUF)], sem_g)

        def wait_g(i):
            pltpu.make_async_copy(
                y_hbm.at[idx_v.at[i]], buf_v.at[lax.rem(i, _NBUF)],
                sem_g).wait()

        def start_w(i):
            pltpu.async_copy(
                buf_v.at[lax.rem(i, _NBUF)],
                out_hbm.at[pl.ds((base_row + i) * _RPC, _RPC)], sem_w)

        def wait_w():
            pltpu.make_async_copy(
                buf_v.at[0], out_hbm.at[pl.ds(base_row * _RPC, _RPC)],
                sem_w).wait()

        for k in range(_NBUF - 1):
            start_g(k)

        def body(i, carry):
            wait_g(i)

            @pl.when(i + (_NBUF - 1) < n_i)
            def _ahead():
                @pl.when(i >= 1)
                def _drain():
                    wait_w()

                start_g(i + (_NBUF - 1))

            start_w(i)
            return carry

        lax.fori_loop(0, n_i, body, 0)
        for _ in range(_NBUF):
            wait_w()

    return gather_k(nbr2, y)


def _tc_main_body(f_ref, g_ref, nm_ref, y_ref, x_ref,
                  wg_ref, w1_ref, b1_ref, w2_ref, b2_ref, w3_ref, b3_ref,
                  wd_ref, bd_ref, mask_ref, o_ref, *, blk, nbh):
    d = y_ref.shape[-1]
    sb = wg_ref.shape[0]
    f2 = f_ref[...].reshape(blk * nbh, sb)
    wf = jnp.dot(f2, wg_ref[...], preferred_element_type=jnp.float32)
    prod = (g_ref[...] * wf).reshape(blk, nbh, d)
    nm = nm_ref[...].reshape(blk, nbh)
    y2 = jnp.sum(prod * nm[..., None], axis=1)
    y = y_ref[...] + y2
    h = y
    for w_r, b_r in ((w1_ref, b1_ref), (w2_ref, b2_ref), (w3_ref, b3_ref)):
        h = _ssp(h)
        h = jnp.dot(h, w_r[...], preferred_element_type=jnp.float32) + b_r[...]
    y = y + h
    y = _ssp(y)
    y = jnp.dot(y, wd_ref[...], preferred_element_type=jnp.float32) + bd_ref[...]
    o_ref[...] = y + mask_ref[...] * x_ref[...]


def _tc_main(f3, G, nm, y, x2, W_G,
             W_res1, b_res1, W_res2, b_res2, W_res3, b_res3,
             W_dense, b_dense, mask, blk):
    n, d = x2.shape
    nbh = f3.shape[1]
    sb = f3.shape[2]
    grid = (n // blk,)
    w_spec = pl.BlockSpec((d, d), lambda i: (0, 0))
    b_spec = pl.BlockSpec((1, d), lambda i: (0, 0))
    return pl.pallas_call(
        functools.partial(_tc_main_body, blk=blk, nbh=nbh),
        grid=grid,
        in_specs=[
            pl.BlockSpec((blk, nbh, sb), lambda i: (i, 0, 0)),
            pl.BlockSpec((blk * nbh, d), lambda i: (i, 0)),
            pl.BlockSpec((1, blk, nbh), lambda i: (0, i, 0)),
            pl.BlockSpec((blk, d), lambda i: (i, 0)),
            pl.BlockSpec((blk, d), lambda i: (i, 0)),
            pl.BlockSpec((sb, d), lambda i: (0, 0)),
            w_spec, b_spec, w_spec, b_spec, w_spec, b_spec,
            w_spec, b_spec, b_spec,
        ],
        out_specs=pl.BlockSpec((blk, d), lambda i: (i, 0)),
        out_shape=jax.ShapeDtypeStruct((n, d), jnp.float32),
    )(f3, G, nm, y, x2, W_G,
      W_res1, b_res1.reshape(1, d), W_res2, b_res2.reshape(1, d),
      W_res3, b_res3.reshape(1, d), W_dense, b_dense.reshape(1, d),
      mask.reshape(1, d))


def kernel(x, r_ij, neighbors, neighbor_mask, f_ij,
           W_in2f, b_in2f, W_G,
           W_res1, b_res1, W_res2, b_res2, W_res3, b_res3,
           W_dense, b_dense, mask):
    b, n, d = x.shape
    nbh = neighbors.shape[-1]
    sb = f_ij.shape[-1]
    n_edges = b * n * nbh

    x2 = x.reshape(b * n, d)
    y = _tc_pre(x2, W_in2f, b_in2f, blk=1000)
    n_rows = n_edges // _RPC
    pad_rows = (-n_rows) % (8 * _NW)
    pad_nodes = pad_rows * _RPC // nbh
    if pad_nodes:
        nbr_pad = jnp.concatenate(
            [neighbors, neighbors[:, :pad_nodes, :]], axis=1)
    else:
        nbr_pad = neighbors
    nbr2 = _tc_repack(nbr_pad)

    G = _sc_gather(nbr2, y, d)

    f3 = f_ij.reshape(b * n, nbh, sb)
    out = _tc_main(f3, G, neighbor_mask, y, x2, W_G,
                   W_res1, b_res1, W_res2, b_res2, W_res3, b_res3,
                   W_dense, b_dense, mask, blk=400)
    return out.reshape(b, n, d)
